# baseline (speedup 1.0000x reference)
import jax
import jax.numpy as jnp
from jax import lax
from jax.experimental import pallas as pl
from jax.experimental.pallas import tpu as pltpu
from jax.experimental.pallas import tpu_sc as plsc

N_PREV = 16384
N_ACT = 16384
ROW_W = 8 * 128
PK_W = 8
NC, NS, L = 2, 16, 16
NW = NC * NS
RPW = N_ACT // NW
VPW = RPW // L
ZROWS = 16
NZ = RPW // ZROWS
WAVE = 16


def _search_compact(prev_v, curr_v, cmp_r, cmp_s, lane):
    def _bs(v, cnt):
        q0 = curr_v[pl.ds(2 * v * L, L)]
        q1 = curr_v[pl.ds((2 * v + 1) * L, L)]
        b0 = jnp.zeros((L,), jnp.int32)
        b1 = jnp.zeros((L,), jnp.int32)
        step = N_PREV // 2
        while step >= 1:
            p0 = plsc.load_gather(prev_v, [b0 + (step - 1)])
            p1 = plsc.load_gather(prev_v, [b1 + (step - 1)])
            b0 = jnp.where(p0 < q0, b0 + step, b0)
            b1 = jnp.where(p1 < q1, b1 + step, b1)
            step //= 2
        l0 = plsc.load_gather(prev_v, [b0])
        l1 = plsc.load_gather(prev_v, [b1])
        pos0 = jnp.minimum(b0 + jnp.where(l0 < q0, 1, 0), N_PREV - 1)
        pos1 = jnp.minimum(b1 + jnp.where(l1 < q1, 1, 0), N_PREV - 1)
        v0 = plsc.load_gather(prev_v, [pos0])
        v1 = plsc.load_gather(prev_v, [pos1])
        f0 = v0 == q0
        f1 = v1 == q1
        plsc.store_compressed(cmp_r.at[pl.ds(cnt, L)], 2 * v * L + lane,
                              mask=f0)
        plsc.store_compressed(cmp_s.at[pl.ds(cnt, L)], pos0, mask=f0)
        cnt = cnt + plsc.all_reduce_population_count(f0)[0]
        plsc.store_compressed(cmp_r.at[pl.ds(cnt, L)], (2 * v + 1) * L + lane,
                              mask=f1)
        plsc.store_compressed(cmp_s.at[pl.ds(cnt, L)], pos1, mask=f1)
        return cnt + plsc.all_reduce_population_count(f1)[0]

    return lax.fori_loop(0, VPW // 2, _bs, 0)


def _small_body(prev_ids, curr_ids, packed_prev,
                occ_out, cert_out, vel_out, mconf_out,
                prev_v, curr_v, cmp_r, cmp_s,
                occ_v, cert_v, vel_v, mconf_v, prow_v,
                sem_prev, sem_curr, sem_s, sem_pk):
    cid = lax.axis_index("c")
    sid = lax.axis_index("s")
    base = (sid * NC + cid) * RPW

    cp_prev = pltpu.async_copy(prev_ids, prev_v, sem_prev)
    cp_curr = pltpu.async_copy(curr_ids.at[pl.ds(base, RPW)], curr_v, sem_curr)

    f32 = jnp.float32

    def _fill(i, c):
        occ_v[pl.ds(i * L, L)] = jnp.zeros((L,), f32)
        cert_v[pl.ds(i * L, L)] = jnp.full((L,), 100.0, f32)
        mconf_v[pl.ds(i * L, L)] = jnp.full((L,), 10.0, f32)
        vel_v[pl.ds(2 * i * L, L)] = jnp.zeros((L,), f32)
        vel_v[pl.ds((2 * i + 1) * L, L)] = jnp.zeros((L,), f32)
        return c

    lax.fori_loop(0, VPW, _fill, 0)

    cp_prev.wait()
    cp_curr.wait()

    lane = lax.iota(jnp.int32, L)
    n_found = _search_compact(prev_v, curr_v, cmp_r, cmp_s, lane)

    def _wave(wv, c):
        w = wv * WAVE
        nw = jnp.minimum(n_found - w, WAVE)

        def _fire(j, c2):
            src = cmp_s[pl.ds(w + j, L)][0]
            pltpu.async_copy(packed_prev.at[pl.ds(src, 1), :],
                             prow_v.at[pl.ds(j, 1), :], sem_pk)
            return c2

        lax.fori_loop(0, nw, _fire, 0)

        def _drain(j, c2):
            pltpu.make_async_copy(packed_prev.at[pl.ds(0, 1), :],
                                  prow_v.at[pl.ds(0, 1), :], sem_pk).wait()
            return c2

        lax.fori_loop(0, nw, _drain, 0)

        def _apply(j, c2):
            r = cmp_r[pl.ds(w + j, L)][0]
            jj = jnp.full((L,), 1, jnp.int32) * j
            pk = plsc.load_gather(prow_v, [jj, jnp.minimum(lane, PK_W - 1)])
            rr = jnp.full((L,), 1, jnp.int32) * r
            plsc.store_scatter(occ_v, [rr], pk, mask=lane == 2)
            plsc.store_scatter(cert_v, [rr], pk, mask=lane == 3)
            plsc.store_scatter(vel_v, [2 * r + jnp.clip(lane - 4, 0, 1)], pk,
                               mask=(lane >= 4) & (lane < 6))
            plsc.store_scatter(mconf_v, [rr], pk, mask=lane == 6)
            return c2

        lax.fori_loop(0, nw, _apply, 0)
        return c

    lax.fori_loop(0, lax.div(n_found + WAVE - 1, WAVE), _wave, 0)

    outs = [
        pltpu.async_copy(occ_v, occ_out.at[pl.ds(base, RPW)], sem_s),
        pltpu.async_copy(cert_v, cert_out.at[pl.ds(base, RPW)], sem_s),
        pltpu.async_copy(vel_v, vel_out.at[pl.ds(2 * base, 2 * RPW)], sem_s),
        pltpu.async_copy(mconf_v, mconf_out.at[pl.ds(base, RPW)], sem_s),
    ]
    for cp in outs:
        cp.wait()


def _pips_body(prev_ids, curr_ids, qpos, packed_prev, zeros_blk, pips_in,
               pips_out, pos_out,
               prev_v, curr_v, cmp_r, cmp_s, zbuf_v, row_v, pos_v, prow_v,
               sem_prev, sem_curr, sem_pos, sem_z, sem_row, sem_wrow,
               sem_pk, sem_s):
    cid = lax.axis_index("c")
    sid = lax.axis_index("s")
    base = (sid * NC + cid) * RPW

    cp_prev = pltpu.async_copy(prev_ids, prev_v, sem_prev)
    cp_curr = pltpu.async_copy(curr_ids.at[pl.ds(base, RPW)], curr_v, sem_curr)
    cp_pos = pltpu.async_copy(qpos.at[0, pl.ds(base, RPW), :], pos_v, sem_pos)
    pltpu.async_copy(zeros_blk, zbuf_v, sem_z).wait()

    zcopies = []
    for z in range(NZ):
        zcopies.append(pltpu.async_copy(
            zbuf_v,
            pips_out.at[pl.ds((base + z * ZROWS) * ROW_W, ZROWS * ROW_W)],
            sem_z))

    cp_prev.wait()
    cp_curr.wait()

    lane = lax.iota(jnp.int32, L)
    n_found = _search_compact(prev_v, curr_v, cmp_r, cmp_s, lane)
    cp_pos.wait()

    for cp in zcopies:
        cp.wait()

    def _wave(wv, c):
        w = wv * WAVE
        nw = jnp.minimum(n_found - w, WAVE)

        def _fire(j, c2):
            src = cmp_s[pl.ds(w + j, L)][0]
            pltpu.async_copy(pips_in.at[pl.ds(src * ROW_W, ROW_W)],
                             row_v.at[pl.ds(j * ROW_W, ROW_W)], sem_row)
            pltpu.async_copy(packed_prev.at[pl.ds(src, 1), :],
                             prow_v.at[pl.ds(j, 1), :], sem_pk)
            return c2

        lax.fori_loop(0, nw, _fire, 0)

        def _drain_g(j, c2):
            pltpu.make_async_copy(pips_in.at[pl.ds(0, ROW_W)],
                                  row_v.at[pl.ds(0, ROW_W)], sem_row).wait()
            pltpu.make_async_copy(packed_prev.at[pl.ds(0, 1), :],
                                  prow_v.at[pl.ds(0, 1), :], sem_pk).wait()
            return c2

        lax.fori_loop(0, nw, _drain_g, 0)

        def _write(j, c2):
            r = cmp_r[pl.ds(w + j, L)][0]
            pltpu.async_copy(
                row_v.at[pl.ds(j * ROW_W, ROW_W)],
                pips_out.at[pl.ds((base + r) * ROW_W, ROW_W)], sem_wrow)
            jj = jnp.full((L,), 1, jnp.int32) * j
            pk = plsc.load_gather(prow_v, [jj, jnp.minimum(lane, PK_W - 1)])
            rr = jnp.full((L,), 1, jnp.int32) * r
            vlane = jnp.minimum(lane, 1)
            plsc.store_scatter(pos_v, [rr, vlane], pk, mask=lane < 2)
            return c2

        lax.fori_loop(0, nw, _write, 0)

        def _drain_w(j, c2):
            pltpu.make_async_copy(pips_in.at[pl.ds(0, ROW_W)],
                                  row_v.at[pl.ds(0, ROW_W)], sem_wrow).wait()
            return c2

        lax.fori_loop(0, nw, _drain_w, 0)
        return c

    lax.fori_loop(0, lax.div(n_found + WAVE - 1, WAVE), _wave, 0)
    pltpu.async_copy(pos_v, pos_out.at[pl.ds(base, RPW), :], sem_s).wait()


def kernel(prev_query_ids, curr_query_ids, query_positions, prev_updated_pos,
           prev_updated_occlusion, prev_updated_certainty,
           prev_updated_velocity, prev_mconf, prev_pips_mem):
    f32 = jnp.float32
    packed_prev = jnp.concatenate([
        prev_updated_pos[0], prev_updated_occlusion[0],
        prev_updated_certainty[0], prev_updated_velocity[0], prev_mconf[0],
        jnp.zeros((N_PREV, 1), f32)], axis=1)
    zeros_blk = jnp.zeros((ZROWS * ROW_W,), f32)
    pips_flat = prev_pips_mem.reshape(-1)

    mesh = plsc.VectorSubcoreMesh(core_axis_name="c", subcore_axis_name="s")
    i32 = jnp.int32

    occ, cert, vel, mconf = pl.kernel(
        _small_body,
        out_type=[
            jax.ShapeDtypeStruct((N_ACT,), f32),
            jax.ShapeDtypeStruct((N_ACT,), f32),
            jax.ShapeDtypeStruct((N_ACT * 2,), f32),
            jax.ShapeDtypeStruct((N_ACT,), f32),
        ],
        mesh=mesh,
        compiler_params=pltpu.CompilerParams(needs_layout_passes=False),
        scratch_types=[
            pltpu.VMEM((N_PREV,), i32),
            pltpu.VMEM((RPW,), i32),
            pltpu.VMEM((RPW + L,), i32),
            pltpu.VMEM((RPW + L,), i32),
            pltpu.VMEM((RPW,), f32),
            pltpu.VMEM((RPW,), f32),
            pltpu.VMEM((2 * RPW,), f32),
            pltpu.VMEM((RPW,), f32),
            pltpu.VMEM((WAVE, PK_W), f32),
            pltpu.SemaphoreType.DMA,
            pltpu.SemaphoreType.DMA,
            pltpu.SemaphoreType.DMA,
            pltpu.SemaphoreType.DMA,
        ],
        name="sc_small_state",
    )(prev_query_ids, curr_query_ids, packed_prev)

    pips, pos = pl.kernel(
        _pips_body,
        out_type=[
            jax.ShapeDtypeStruct((N_ACT * ROW_W,), f32),
            jax.ShapeDtypeStruct((N_ACT, 2), f32),
        ],
        mesh=mesh,
        compiler_params=pltpu.CompilerParams(needs_layout_passes=False),
        scratch_types=[
            pltpu.VMEM((N_PREV,), i32),
            pltpu.VMEM((RPW,), i32),
            pltpu.VMEM((RPW + L,), i32),
            pltpu.VMEM((RPW + L,), i32),
            pltpu.VMEM((ZROWS * ROW_W,), f32),
            pltpu.VMEM((WAVE * ROW_W,), f32),
            pltpu.VMEM((RPW, 2), f32),
            pltpu.VMEM((WAVE, PK_W), f32),
            pltpu.SemaphoreType.DMA,
            pltpu.SemaphoreType.DMA,
            pltpu.SemaphoreType.DMA,
            pltpu.SemaphoreType.DMA,
            pltpu.SemaphoreType.DMA,
            pltpu.SemaphoreType.DMA,
            pltpu.SemaphoreType.DMA,
            pltpu.SemaphoreType.DMA,
        ],
        name="sc_pips",
    )(prev_query_ids, curr_query_ids, query_positions, packed_prev,
      zeros_blk, pips_flat)

    return (pos[None], occ.reshape(1, N_ACT, 1),
            cert.reshape(1, N_ACT, 1), vel.reshape(1, N_ACT, 2),
            mconf.reshape(1, N_ACT, 1), pips.reshape(N_ACT, 8, 128))


# speedup vs baseline: 1.1332x; 1.1332x over previous
import jax
import jax.numpy as jnp
from jax import lax
from jax.experimental import pallas as pl
from jax.experimental.pallas import tpu as pltpu
from jax.experimental.pallas import tpu_sc as plsc

N_PREV = 16384
N_ACT = 16384
ROW_W = 8 * 128
PK_W = 8
NC, NS, L = 2, 16, 16
NW = NC * NS
RPW = N_ACT // NW
VPW = RPW // L
ZROWS = 32
NZ = RPW // ZROWS
WAVE = 16


def _search_compact(prev_v, curr_v, cmp_r, cmp_s, lane):
    def _bs(v, cnt):
        q0 = curr_v[pl.ds(2 * v * L, L)]
        q1 = curr_v[pl.ds((2 * v + 1) * L, L)]
        b0 = jnp.zeros((L,), jnp.int32)
        b1 = jnp.zeros((L,), jnp.int32)
        step = N_PREV // 2
        while step >= 1:
            p0 = plsc.load_gather(prev_v, [b0 + (step - 1)])
            p1 = plsc.load_gather(prev_v, [b1 + (step - 1)])
            b0 = jnp.where(p0 < q0, b0 + step, b0)
            b1 = jnp.where(p1 < q1, b1 + step, b1)
            step //= 2
        l0 = plsc.load_gather(prev_v, [b0])
        l1 = plsc.load_gather(prev_v, [b1])
        pos0 = jnp.minimum(b0 + jnp.where(l0 < q0, 1, 0), N_PREV - 1)
        pos1 = jnp.minimum(b1 + jnp.where(l1 < q1, 1, 0), N_PREV - 1)
        v0 = plsc.load_gather(prev_v, [pos0])
        v1 = plsc.load_gather(prev_v, [pos1])
        f0 = v0 == q0
        f1 = v1 == q1
        plsc.store_compressed(cmp_r.at[pl.ds(cnt, L)], 2 * v * L + lane,
                              mask=f0)
        plsc.store_compressed(cmp_s.at[pl.ds(cnt, L)], pos0, mask=f0)
        cnt = cnt + plsc.all_reduce_population_count(f0)[0]
        plsc.store_compressed(cmp_r.at[pl.ds(cnt, L)], (2 * v + 1) * L + lane,
                              mask=f1)
        plsc.store_compressed(cmp_s.at[pl.ds(cnt, L)], pos1, mask=f1)
        return cnt + plsc.all_reduce_population_count(f1)[0]

    return lax.fori_loop(0, VPW // 2, _bs, 0)


def _small_body(prev_ids, curr_ids, qpos, packed_prev,
                pos_out, occ_out, cert_out, vel_out, mconf_out,
                prev_v, curr_v, cmp_r, cmp_s,
                pos_v, occ_v, cert_v, vel_v, mconf_v, prow_v,
                sem_prev, sem_curr, sem_pos, sem_s, sem_pk):
    cid = lax.axis_index("c")
    sid = lax.axis_index("s")
    base = (sid * NC + cid) * RPW

    cp_prev = pltpu.async_copy(prev_ids, prev_v, sem_prev)
    cp_curr = pltpu.async_copy(curr_ids.at[pl.ds(base, RPW)], curr_v, sem_curr)
    cp_pos = pltpu.async_copy(qpos.at[0, pl.ds(base, RPW), :], pos_v, sem_pos)

    f32 = jnp.float32

    def _fill(i, c):
        occ_v[pl.ds(i * L, L)] = jnp.zeros((L,), f32)
        cert_v[pl.ds(i * L, L)] = jnp.full((L,), 100.0, f32)
        mconf_v[pl.ds(i * L, L)] = jnp.full((L,), 10.0, f32)
        vel_v[pl.ds(2 * i * L, L)] = jnp.zeros((L,), f32)
        vel_v[pl.ds((2 * i + 1) * L, L)] = jnp.zeros((L,), f32)
        return c

    lax.fori_loop(0, VPW, _fill, 0)

    cp_prev.wait()
    cp_curr.wait()

    lane = lax.iota(jnp.int32, L)
    n_found = _search_compact(prev_v, curr_v, cmp_r, cmp_s, lane)
    cp_pos.wait()

    def _wave(wv, c):
        w = wv * WAVE
        nw = jnp.minimum(n_found - w, WAVE)

        def _fire(j, c2):
            src = cmp_s[pl.ds(w + j, L)][0]
            pltpu.async_copy(packed_prev.at[pl.ds(src, 1), :],
                             prow_v.at[pl.ds(j, 1), :], sem_pk)
            return c2

        lax.fori_loop(0, nw, _fire, 0)

        def _drain(j, c2):
            pltpu.make_async_copy(packed_prev.at[pl.ds(0, 1), :],
                                  prow_v.at[pl.ds(0, 1), :], sem_pk).wait()
            return c2

        lax.fori_loop(0, nw, _drain, 0)

        def _apply(j, c2):
            r = cmp_r[pl.ds(w + j, L)][0]
            jj = jnp.full((L,), 1, jnp.int32) * j
            pk = plsc.load_gather(prow_v, [jj, jnp.minimum(lane, PK_W - 1)])
            rr = jnp.full((L,), 1, jnp.int32) * r
            vlane = jnp.minimum(lane, 1)
            plsc.store_scatter(pos_v, [rr, vlane], pk, mask=lane < 2)
            plsc.store_scatter(occ_v, [rr], pk, mask=lane == 2)
            plsc.store_scatter(cert_v, [rr], pk, mask=lane == 3)
            plsc.store_scatter(vel_v, [2 * r + jnp.clip(lane - 4, 0, 1)], pk,
                               mask=(lane >= 4) & (lane < 6))
            plsc.store_scatter(mconf_v, [rr], pk, mask=lane == 6)
            return c2

        lax.fori_loop(0, nw, _apply, 0)
        return c

    lax.fori_loop(0, lax.div(n_found + WAVE - 1, WAVE), _wave, 0)

    outs = [
        pltpu.async_copy(pos_v, pos_out.at[pl.ds(base, RPW), :], sem_s),
        pltpu.async_copy(occ_v, occ_out.at[pl.ds(base, RPW)], sem_s),
        pltpu.async_copy(cert_v, cert_out.at[pl.ds(base, RPW)], sem_s),
        pltpu.async_copy(vel_v, vel_out.at[pl.ds(2 * base, 2 * RPW)], sem_s),
        pltpu.async_copy(mconf_v, mconf_out.at[pl.ds(base, RPW)], sem_s),
    ]
    for cp in outs:
        cp.wait()


def _pips_body(prev_ids, curr_ids, zeros_blk, pips_in, pips_out,
               prev_v, curr_v, cmp_r, cmp_s, zbuf_v, row_v,
               sem_prev, sem_curr, sem_z, sem_row, sem_wrow):
    cid = lax.axis_index("c")
    sid = lax.axis_index("s")
    base = (sid * NC + cid) * RPW

    cp_prev = pltpu.async_copy(prev_ids, prev_v, sem_prev)
    cp_curr = pltpu.async_copy(curr_ids.at[pl.ds(base, RPW)], curr_v, sem_curr)
    pltpu.async_copy(zeros_blk, zbuf_v, sem_z).wait()

    zcopies = []
    for z in range(NZ):
        zcopies.append(pltpu.async_copy(
            zbuf_v,
            pips_out.at[pl.ds((base + z * ZROWS) * ROW_W, ZROWS * ROW_W)],
            sem_z))

    cp_prev.wait()
    cp_curr.wait()

    lane = lax.iota(jnp.int32, L)
    n_found = _search_compact(prev_v, curr_v, cmp_r, cmp_s, lane)

    for cp in zcopies:
        cp.wait()

    def _wave(wv, c):
        w = wv * WAVE
        nw = jnp.minimum(n_found - w, WAVE)

        def _fire(j, c2):
            src = cmp_s[pl.ds(w + j, L)][0]
            pltpu.async_copy(pips_in.at[pl.ds(src * ROW_W, ROW_W)],
                             row_v.at[pl.ds(j * ROW_W, ROW_W)], sem_row)
            return c2

        lax.fori_loop(0, nw, _fire, 0)

        def _drain_g(j, c2):
            pltpu.make_async_copy(pips_in.at[pl.ds(0, ROW_W)],
                                  row_v.at[pl.ds(0, ROW_W)], sem_row).wait()
            return c2

        lax.fori_loop(0, nw, _drain_g, 0)

        def _write(j, c2):
            r = cmp_r[pl.ds(w + j, L)][0]
            pltpu.async_copy(
                row_v.at[pl.ds(j * ROW_W, ROW_W)],
                pips_out.at[pl.ds((base + r) * ROW_W, ROW_W)], sem_wrow)
            return c2

        lax.fori_loop(0, nw, _write, 0)

        def _drain_w(j, c2):
            pltpu.make_async_copy(pips_in.at[pl.ds(0, ROW_W)],
                                  row_v.at[pl.ds(0, ROW_W)], sem_wrow).wait()
            return c2

        lax.fori_loop(0, nw, _drain_w, 0)
        return c

    lax.fori_loop(0, lax.div(n_found + WAVE - 1, WAVE), _wave, 0)


def kernel(prev_query_ids, curr_query_ids, query_positions, prev_updated_pos,
           prev_updated_occlusion, prev_updated_certainty,
           prev_updated_velocity, prev_mconf, prev_pips_mem):
    f32 = jnp.float32
    packed_prev = jnp.concatenate([
        prev_updated_pos[0], prev_updated_occlusion[0],
        prev_updated_certainty[0], prev_updated_velocity[0], prev_mconf[0],
        jnp.zeros((N_PREV, 1), f32)], axis=1)
    zeros_blk = jnp.zeros((ZROWS * ROW_W,), f32)
    pips_flat = prev_pips_mem.reshape(-1)

    mesh = plsc.VectorSubcoreMesh(core_axis_name="c", subcore_axis_name="s")
    i32 = jnp.int32

    pos, occ, cert, vel, mconf = pl.kernel(
        _small_body,
        out_type=[
            jax.ShapeDtypeStruct((N_ACT, 2), f32),
            jax.ShapeDtypeStruct((N_ACT,), f32),
            jax.ShapeDtypeStruct((N_ACT,), f32),
            jax.ShapeDtypeStruct((N_ACT * 2,), f32),
            jax.ShapeDtypeStruct((N_ACT,), f32),
        ],
        mesh=mesh,
        compiler_params=pltpu.CompilerParams(needs_layout_passes=False),
        scratch_types=[
            pltpu.VMEM((N_PREV,), i32),
            pltpu.VMEM((RPW,), i32),
            pltpu.VMEM((RPW + L,), i32),
            pltpu.VMEM((RPW + L,), i32),
            pltpu.VMEM((RPW, 2), f32),
            pltpu.VMEM((RPW,), f32),
            pltpu.VMEM((RPW,), f32),
            pltpu.VMEM((2 * RPW,), f32),
            pltpu.VMEM((RPW,), f32),
            pltpu.VMEM((WAVE, PK_W), f32),
            pltpu.SemaphoreType.DMA,
            pltpu.SemaphoreType.DMA,
            pltpu.SemaphoreType.DMA,
            pltpu.SemaphoreType.DMA,
            pltpu.SemaphoreType.DMA,
        ],
        name="sc_small_state",
    )(prev_query_ids, curr_query_ids, query_positions, packed_prev)

    pips = pl.kernel(
        _pips_body,
        out_type=jax.ShapeDtypeStruct((N_ACT * ROW_W,), f32),
        mesh=mesh,
        compiler_params=pltpu.CompilerParams(needs_layout_passes=False),
        scratch_types=[
            pltpu.VMEM((N_PREV,), i32),
            pltpu.VMEM((RPW,), i32),
            pltpu.VMEM((RPW + L,), i32),
            pltpu.VMEM((RPW + L,), i32),
            pltpu.VMEM((ZROWS * ROW_W,), f32),
            pltpu.VMEM((WAVE * ROW_W,), f32),
            pltpu.SemaphoreType.DMA,
            pltpu.SemaphoreType.DMA,
            pltpu.SemaphoreType.DMA,
            pltpu.SemaphoreType.DMA,
            pltpu.SemaphoreType.DMA,
        ],
        name="sc_pips",
    )(prev_query_ids, curr_query_ids, zeros_blk, pips_flat)

    return (pos[None], occ.reshape(1, N_ACT, 1),
            cert.reshape(1, N_ACT, 1), vel.reshape(1, N_ACT, 2),
            mconf.reshape(1, N_ACT, 1), pips.reshape(N_ACT, 8, 128))
